# baseline (speedup 1.0000x reference)
import functools
import math

import jax
import jax.numpy as jnp
from jax import lax
from jax.experimental import pallas as pl
from jax.experimental.pallas import tpu as pltpu
from jax.experimental.pallas import tpu_sc as plsc

D_EMB = 64
D_PAD = 128
SCALE = 1.0 / math.sqrt(D_EMB)

NUM_CORES = 2
NUM_SUBCORES = 16
NW = NUM_CORES * NUM_SUBCORES

B_TOTAL = 4096 * 200
B_PER_W = B_TOTAL // NW
CHUNK = 128
N_CHUNKS = B_PER_W // CHUNK
K = 4
N_GROUPS = N_CHUNKS // K


def _emb_body(x_hbm, table_hbm, out_hbm, idx_all, *bufs):
    gbuf = bufs[0:K]
    sbuf = bufs[K:2 * K]
    gsem = bufs[2 * K:3 * K]
    ssem = bufs[3 * K:4 * K]

    wid = lax.axis_index("s") * NUM_CORES + lax.axis_index("c")
    base = wid * B_PER_W

    pltpu.sync_copy(x_hbm.at[pl.ds(base, B_PER_W)], idx_all)

    def gather_copy(g, b):
        idx_c = idx_all.at[pl.ds(g * CHUNK, CHUNK)]
        return pltpu.make_async_copy(table_hbm.at[idx_c], gbuf[b], gsem[b])

    def store_copy(g, b):
        dst = out_hbm.at[pl.ds(base + g * CHUNK, CHUNK), pl.ds(0, D_EMB)]
        return pltpu.make_async_copy(sbuf[b], dst, ssem[b])

    for b in range(K):
        gather_copy(b, b).start()

    def group_body(p, carry):
        for b in range(K):
            g = p * K + b
            gather_copy(g, b).wait()

            @pl.when(p > 0)
            def _():
                store_copy(g - K, b).wait()

            def row_body(r, c):
                for j in range(D_EMB // 16):
                    sl = pl.ds(j * 16, 16)
                    sbuf[b][r, sl] = gbuf[b][r, sl] * SCALE
                return c

            lax.fori_loop(0, CHUNK, row_body, 0)

            @pl.when(g + K < N_CHUNKS)
            def _():
                gather_copy(g + K, b).start()

            store_copy(g, b).start()
        return carry

    lax.fori_loop(0, N_GROUPS, group_body, 0)

    for b in range(K):
        store_copy(N_CHUNKS - K + b, b).wait()


def kernel(x, table):
    b, s = x.shape
    flat_x = x.reshape((b * s,)).astype(jnp.int32)
    padded = jnp.pad(table, ((0, 0), (0, D_PAD - D_EMB)))

    scratch = (
        [pltpu.VMEM((B_PER_W,), jnp.int32)]
        + [pltpu.VMEM((CHUNK, D_PAD), jnp.float32) for _ in range(K)]
        + [pltpu.VMEM((CHUNK, D_EMB), jnp.float32) for _ in range(K)]
        + [pltpu.SemaphoreType.DMA for _ in range(2 * K)]
    )
    mesh = plsc.VectorSubcoreMesh(core_axis_name="c", subcore_axis_name="s")
    emb = functools.partial(
        pl.kernel,
        mesh=mesh,
        out_type=jax.ShapeDtypeStruct((B_TOTAL, D_PAD), jnp.float32),
        scratch_types=scratch,
        compiler_params=pltpu.CompilerParams(use_tc_tiling_on_sc=False),
    )(_emb_body)

    out = emb(flat_x, padded)
    return out[:, :D_EMB].reshape((b, s, D_EMB))


# speedup vs baseline: 1.1891x; 1.1891x over previous
import functools
import math

import jax
import jax.numpy as jnp
from jax import lax
from jax.experimental import pallas as pl
from jax.experimental.pallas import tpu as pltpu
from jax.experimental.pallas import tpu_sc as plsc

D_EMB = 64
D_PAD = 128
SCALE = 1.0 / math.sqrt(D_EMB)

NUM_CORES = 2
NUM_SUBCORES = 16
NW = NUM_CORES * NUM_SUBCORES

B_TOTAL = 4096 * 200
B_PER_W = B_TOTAL // NW
CHUNK = 128
N_CHUNKS = B_PER_W // CHUNK
K = 5
N_GROUPS = N_CHUNKS // K


def _emb_body(x_hbm, table_hbm, out_hbm, idx_all, *bufs):
    gbuf = bufs[0:K]
    gsem = bufs[K:2 * K]
    ssem = bufs[2 * K:3 * K]

    wid = lax.axis_index("s") * NUM_CORES + lax.axis_index("c")
    base = wid * B_PER_W

    pltpu.sync_copy(x_hbm.at[pl.ds(base, B_PER_W)], idx_all)

    def gather_copy(g, b):
        idx_c = idx_all.at[pl.ds(g * CHUNK, CHUNK)]
        return pltpu.make_async_copy(table_hbm.at[idx_c], gbuf[b], gsem[b])

    def store_copy(g, b):
        dst = out_hbm.at[pl.ds(base + g * CHUNK, CHUNK)]
        return pltpu.make_async_copy(gbuf[b], dst, ssem[b])

    for b in range(K):
        gather_copy(b, b).start()

    def group_body(p, carry):
        for b in range(K):
            g = p * K + b
            gather_copy(g, b).wait()

            def row_body(r, c):
                for j in range(D_EMB // 16):
                    sl = pl.ds(j * 16, 16)
                    gbuf[b][r, sl] = gbuf[b][r, sl] * SCALE
                return c

            lax.fori_loop(0, CHUNK, row_body, 0)

            store_copy(g, b).start()

            @pl.when(g + K < N_CHUNKS)
            def _():
                store_copy(g, b).wait()
                gather_copy(g + K, b).start()
        return carry

    lax.fori_loop(0, N_GROUPS, group_body, 0)

    for b in range(K):
        store_copy(N_CHUNKS - K + b, b).wait()


def kernel(x, table):
    b, s = x.shape
    flat_x = x.reshape((b * s,)).astype(jnp.int32)
    padded = jnp.pad(table, ((0, 0), (0, D_PAD - D_EMB)))

    scratch = (
        [pltpu.VMEM((B_PER_W,), jnp.int32)]
        + [pltpu.VMEM((CHUNK, D_PAD), jnp.float32) for _ in range(K)]
        + [pltpu.SemaphoreType.DMA for _ in range(2 * K)]
    )
    mesh = plsc.VectorSubcoreMesh(core_axis_name="c", subcore_axis_name="s")
    emb = functools.partial(
        pl.kernel,
        mesh=mesh,
        out_type=jax.ShapeDtypeStruct((B_TOTAL, D_PAD), jnp.float32),
        scratch_types=scratch,
        compiler_params=pltpu.CompilerParams(use_tc_tiling_on_sc=False),
    )(_emb_body)

    out = emb(flat_x, padded)
    return out[:, :D_EMB].reshape((b, s, D_EMB))
